# baseline (speedup 1.0000x reference)
import functools
import itertools

import jax
import jax.numpy as jnp
import numpy as np
from jax.experimental import pallas as pl
from jax.experimental.pallas import tpu as pltpu


@functools.lru_cache(maxsize=None)
def _connection_mask(board_size):
    dims = len(board_size)
    dirs = [d for d in itertools.product((-1, 0, 1), repeat=dims)
            if any(x != 0 for x in d)]
    strides = []
    s = 1
    for D in reversed(board_size):
        strides.append(s)
        s *= D
    strides = strides[::-1]
    N = s
    mask = np.zeros((N, N), dtype=np.uint8)
    for fi, idx in enumerate(itertools.product(*(range(D) for D in board_size))):
        for d in dirs:
            t = 1
            while True:
                n = tuple(i + t * di for i, di in zip(idx, d))
                if all(0 <= j < D for j, D in zip(n, board_size)):
                    mask[fi, sum(j * st for j, st in zip(n, strides))] = 1
                    t += 1
                else:
                    break
    return mask


def _attn_kernel(xq_ref, xk_ref, xv_ref, wq_ref, bq_ref, wk_ref, bk_ref,
                 wv_ref, bv_ref, mask_ref, out_ref, k_s, v_s, *, scale, rn):
    rb = pl.program_id(1)

    @pl.when(rb == 0)
    def _project_kv():
        k_s[...] = jax.lax.dot(xk_ref[0], wk_ref[...],
                               preferred_element_type=jnp.float32) + bk_ref[...]
        v_s[...] = jax.lax.dot(xv_ref[0], wv_ref[...],
                               preferred_element_type=jnp.float32) + bv_ref[...]

    q = jax.lax.dot(xq_ref[0], wq_ref[...],
                    preferred_element_type=jnp.float32) + bq_ref[...]
    s = jax.lax.dot_general(q, k_s[...], (((1,), (1,)), ((), ())),
                            preferred_element_type=jnp.float32) * scale
    msk = mask_ref[pl.ds(rb * rn, rn), :]
    s = jnp.where(msk != 0, s, -1e30)
    m = jnp.max(s, axis=1, keepdims=True)
    e = jnp.exp(s - m)
    denom = jnp.sum(e, axis=1, keepdims=True)
    inv = jnp.reciprocal(denom)
    inv = inv * (2.0 - denom * inv)
    att = e * inv
    out_ref[0] = jax.lax.dot(att, v_s[...], preferred_element_type=jnp.float32)


def kernel(query_X, key_X, value_X, Wq, bq, Wk, bk, Wv, bv):
    B = query_X.shape[0]
    board = tuple(int(d) for d in query_X.shape[1:-1])
    in_dim = query_X.shape[-1]
    cmp_dim = Wq.shape[1]
    out_dim = Wv.shape[1]
    mask = jnp.asarray(_connection_mask(board))
    N = mask.shape[0]
    NRB = 6
    RN = N // NRB

    xq = query_X.reshape(B, N, in_dim)
    xk = key_X.reshape(B, N, in_dim)
    xv = value_X.reshape(B, N, in_dim)

    qmap = lambda b, rb: (b, rb, 0)
    bmap = lambda b, rb: (b, 0, 0)
    cmap = lambda b, rb: (0, 0)
    grid = (B, NRB)
    in_specs = [
            pl.BlockSpec((1, RN, in_dim), qmap),
            pl.BlockSpec((1, N, in_dim), bmap),
            pl.BlockSpec((1, N, in_dim), bmap),
            pl.BlockSpec((in_dim, cmp_dim), cmap),
            pl.BlockSpec((1, cmp_dim), cmap),
            pl.BlockSpec((in_dim, cmp_dim), cmap),
            pl.BlockSpec((1, cmp_dim), cmap),
            pl.BlockSpec((in_dim, out_dim), cmap),
            pl.BlockSpec((1, out_dim), cmap),
            pl.BlockSpec((N, N), cmap),
    ]
    out = pl.pallas_call(
        functools.partial(_attn_kernel, scale=1.0 / (cmp_dim ** 0.5), rn=RN),
        grid=grid,
        in_specs=in_specs,
        out_specs=pl.BlockSpec((1, RN, out_dim), qmap),
        out_shape=jax.ShapeDtypeStruct((B, N, out_dim), jnp.float32),
        scratch_shapes=[
            pltpu.VMEM((N, cmp_dim), jnp.float32),
            pltpu.VMEM((N, out_dim), jnp.float32),
        ],
    )(xq, xk, xv, Wq, bq.reshape(1, cmp_dim), Wk, bk.reshape(1, cmp_dim),
      Wv, bv.reshape(1, out_dim), mask)
    return out.reshape((B,) + board + (out_dim,))


# speedup vs baseline: 1.2678x; 1.2678x over previous
import functools
import itertools

import jax
import jax.numpy as jnp
import numpy as np
from jax.experimental import pallas as pl


@functools.lru_cache(maxsize=None)
def _connection_mask(board_size):
    dims = len(board_size)
    dirs = [d for d in itertools.product((-1, 0, 1), repeat=dims)
            if any(x != 0 for x in d)]
    strides = []
    s = 1
    for D in reversed(board_size):
        strides.append(s)
        s *= D
    strides = strides[::-1]
    N = s
    mask = np.zeros((N, N), dtype=np.uint8)
    for fi, idx in enumerate(itertools.product(*(range(D) for D in board_size))):
        for d in dirs:
            t = 1
            while True:
                n = tuple(i + t * di for i, di in zip(idx, d))
                if all(0 <= j < D for j, D in zip(n, board_size)):
                    mask[fi, sum(j * st for j, st in zip(n, strides))] = 1
                    t += 1
                else:
                    break
    return mask


def _attn_kernel(xq_ref, xk_ref, xv_ref, wq_ref, bq_ref, wk_ref, bk_ref,
                 wv_ref, bv_ref, mask_ref, out_ref, *, scale, n, board):
    in_dim = xq_ref.shape[-1]
    xq = jnp.reshape(xq_ref[0], (n, in_dim))
    xk = jnp.reshape(xk_ref[0], (n, in_dim))
    xv = jnp.reshape(xv_ref[0], (n, in_dim))
    q = jax.lax.dot(xq, wq_ref[...],
                    preferred_element_type=jnp.float32) + bq_ref[...]
    k = jax.lax.dot(xk, wk_ref[...],
                    preferred_element_type=jnp.float32) + bk_ref[...]
    v = jax.lax.dot(xv, wv_ref[...],
                    preferred_element_type=jnp.float32) + bv_ref[...]
    s = jax.lax.dot_general(q, k, (((1,), (1,)), ((), ())),
                            preferred_element_type=jnp.float32) * scale
    s = jnp.where(mask_ref[...] != 0, s, -1e30)
    m = jnp.max(s, axis=1, keepdims=True)
    e = jnp.exp(s - m)
    denom = jnp.sum(e, axis=1, keepdims=True)
    inv = jnp.reciprocal(denom)
    inv = inv * (2.0 - denom * inv)
    att = e * inv
    out = jax.lax.dot(att, v, preferred_element_type=jnp.float32)
    out_ref[0] = jnp.reshape(out, board + (out.shape[-1],))


def kernel(query_X, key_X, value_X, Wq, bq, Wk, bk, Wv, bv):
    B = query_X.shape[0]
    board = tuple(int(d) for d in query_X.shape[1:-1])
    in_dim = query_X.shape[-1]
    cmp_dim = Wq.shape[1]
    out_dim = Wv.shape[1]
    mask = jnp.asarray(_connection_mask(board))
    N = mask.shape[0]

    nb = len(board)
    xmap = lambda b: (b,) + (0,) * (nb + 1)
    cmap = lambda b: (0, 0)
    xspec = pl.BlockSpec((1,) + board + (in_dim,), xmap)
    in_specs = [
        xspec,
        xspec,
        xspec,
        pl.BlockSpec((in_dim, cmp_dim), cmap),
        pl.BlockSpec((1, cmp_dim), cmap),
        pl.BlockSpec((in_dim, cmp_dim), cmap),
        pl.BlockSpec((1, cmp_dim), cmap),
        pl.BlockSpec((in_dim, out_dim), cmap),
        pl.BlockSpec((1, out_dim), cmap),
        pl.BlockSpec((N, N), cmap),
    ]
    out = pl.pallas_call(
        functools.partial(_attn_kernel, scale=1.0 / (cmp_dim ** 0.5),
                          n=N, board=board),
        grid=(B,),
        in_specs=in_specs,
        out_specs=pl.BlockSpec((1,) + board + (out_dim,), xmap),
        out_shape=jax.ShapeDtypeStruct((B,) + board + (out_dim,), jnp.float32),
    )(query_X, key_X, value_X, Wq, bq.reshape(1, cmp_dim), Wk,
      bk.reshape(1, cmp_dim), Wv, bv.reshape(1, out_dim), mask)
    return out
